# Initial kernel scaffold; baseline (speedup 1.0000x reference)
import functools

import jax
import jax.numpy as jnp
from jax import lax
from jax.experimental import pallas as pl
from jax.experimental.pallas import tpu as pltpu
from jax.experimental.pallas import tpu_sc as plsc

N_ATOMS = 50000
D_OUT = 384
CH = 200
NUM_CHUNKS = N_ATOMS // CH


def _make_kernel(nc: int, ns: int):
    nw = nc * ns
    cpw = -(-NUM_CHUNKS // nw)

    mesh = plsc.VectorSubcoreMesh(core_axis_name="c", subcore_axis_name="s")

    @functools.partial(
        pl.kernel,
        mesh=mesh,
        out_type=jax.ShapeDtypeStruct((N_ATOMS, D_OUT), jnp.float32),
        scratch_types=[
            pltpu.VMEM((CH,), jnp.int32),
            pltpu.VMEM((CH,), jnp.int32),
            pltpu.VMEM((CH,), jnp.int32),
            pltpu.VMEM((CH,), jnp.int32),
            pltpu.VMEM((CH, D_OUT), jnp.float32),
            pltpu.SemaphoreType.DMA,
        ],
    )
    def k(i1_hbm, i2_hbm, i3_hbm, i4_hbm, t1_hbm, t2_hbm, t3_hbm, t4_hbm,
          out_hbm, i1_v, i2_v, i3_v, i4_v, comb_v, sem):
        wid = lax.axis_index("s") * nc + lax.axis_index("c")

        def chunk_body(kk, carry):
            c = wid + kk * nw

            @pl.when(c < NUM_CHUNKS)
            def _():
                base = c * CH
                a = pltpu.async_copy(i1_hbm.at[pl.ds(base, CH)], i1_v, sem)
                b = pltpu.async_copy(i2_hbm.at[pl.ds(base, CH)], i2_v, sem)
                d = pltpu.async_copy(i3_hbm.at[pl.ds(base, CH)], i3_v, sem)
                e = pltpu.async_copy(i4_hbm.at[pl.ds(base, CH)], i4_v, sem)
                a.wait(); b.wait(); d.wait(); e.wait()
                g1 = pltpu.async_copy(t1_hbm.at[i1_v], comb_v.at[:, pl.ds(0, 128)], sem)
                g2 = pltpu.async_copy(t2_hbm.at[i2_v], comb_v.at[:, pl.ds(128, 128)], sem)
                g3 = pltpu.async_copy(t3_hbm.at[i3_v], comb_v.at[:, pl.ds(256, 64)], sem)
                g4 = pltpu.async_copy(t4_hbm.at[i4_v], comb_v.at[:, pl.ds(320, 64)], sem)
                g1.wait(); g2.wait(); g3.wait(); g4.wait()
                pltpu.sync_copy(comb_v, out_hbm.at[pl.ds(base, CH)])

            return carry

        lax.fori_loop(0, cpw, chunk_body, 0)

    return k


def kernel(atom_type_index, atom_code_index, residue_code_index,
           residue_sequence_index, atom_type_table, atom_code_table,
           residue_code_table, residue_index_table):
    info = plsc.get_sparse_core_info()
    k = _make_kernel(info.num_cores, info.num_subcores)
    return k(atom_type_index.astype(jnp.int32),
             atom_code_index.astype(jnp.int32),
             residue_code_index.astype(jnp.int32),
             residue_sequence_index.astype(jnp.int32),
             atom_type_table, atom_code_table,
             residue_code_table, residue_index_table)


# speedup vs baseline: 1.9192x; 1.9192x over previous
import functools

import jax
import jax.numpy as jnp
from jax import lax
from jax.experimental import pallas as pl
from jax.experimental.pallas import tpu as pltpu
from jax.experimental.pallas import tpu_sc as plsc

N_ATOMS = 50000
D_OUT = 384
CH = 200
NUM_CHUNKS = N_ATOMS // CH


def _make_kernel(nc: int, ns: int):
    nw = nc * ns
    cpw = -(-NUM_CHUNKS // nw)

    mesh = plsc.VectorSubcoreMesh(core_axis_name="c", subcore_axis_name="s")

    @functools.partial(
        pl.kernel,
        mesh=mesh,
        out_type=jax.ShapeDtypeStruct((N_ATOMS, D_OUT), jnp.float32),
        scratch_types=[
            pltpu.VMEM((CH,), jnp.int32),
            pltpu.VMEM((CH,), jnp.int32),
            pltpu.VMEM((CH,), jnp.int32),
            pltpu.VMEM((CH,), jnp.int32),
            pltpu.VMEM((CH, D_OUT), jnp.float32),
            pltpu.VMEM((CH, 128), jnp.float32),
            pltpu.SemaphoreType.DMA,
        ],
    )
    def k(i1_hbm, i2_hbm, i3_hbm, i4_hbm, t1_hbm, t2_hbm, t3_hbm, t4_hbm,
          out_hbm, i1_v, i2_v, i3_v, i4_v, comb_v, buf4_v, sem):
        wid = lax.axis_index("s") * nc + lax.axis_index("c")

        def chunk_body(kk, carry):
            c = wid + kk * nw

            @pl.when(c < NUM_CHUNKS)
            def _():
                base = c * CH
                a = pltpu.async_copy(i1_hbm.at[pl.ds(base, CH)], i1_v, sem)
                b = pltpu.async_copy(i2_hbm.at[pl.ds(base, CH)], i2_v, sem)
                d = pltpu.async_copy(i3_hbm.at[pl.ds(base, CH)], i3_v, sem)
                e = pltpu.async_copy(i4_hbm.at[pl.ds(base, CH)], i4_v, sem)
                a.wait(); b.wait(); d.wait(); e.wait()
                g1 = pltpu.async_copy(t1_hbm.at[i1_v], comb_v.at[:, pl.ds(0, 128)], sem)
                g2 = pltpu.async_copy(t2_hbm.at[i2_v], comb_v.at[:, pl.ds(128, 128)], sem)
                g3 = pltpu.async_copy(t3_hbm.at[i3_v], comb_v.at[:, pl.ds(256, 128)], sem)
                g4 = pltpu.async_copy(t4_hbm.at[i4_v], buf4_v, sem)
                g1.wait(); g2.wait(); g3.wait(); g4.wait()

                def copy_row(r, cc):
                    for s in range(4):
                        comb_v[r, pl.ds(320 + 16 * s, 16)] = buf4_v[r, pl.ds(16 * s, 16)]
                    return cc

                lax.fori_loop(0, CH, copy_row, 0)
                pltpu.sync_copy(comb_v, out_hbm.at[pl.ds(base, CH)])

            return carry

        lax.fori_loop(0, cpw, chunk_body, 0)

    return k


def kernel(atom_type_index, atom_code_index, residue_code_index,
           residue_sequence_index, atom_type_table, atom_code_table,
           residue_code_table, residue_index_table):
    i1 = atom_type_index.astype(jnp.int32)
    i2 = atom_code_index.astype(jnp.int32)
    i3 = residue_code_index.astype(jnp.int32)
    i4 = residue_sequence_index.astype(jnp.int32)
    t3p = jnp.pad(residue_code_table, ((0, 0), (0, 64)))
    t4p = jnp.pad(residue_index_table, ((0, 0), (0, 64)))
    info = plsc.get_sparse_core_info()
    k = _make_kernel(info.num_cores, info.num_subcores)
    return k(i1, i2, i3, i4, atom_type_table, atom_code_table, t3p, t4p)
